# baseline (speedup 1.0000x reference)
import numpy as np
import jax
import jax.numpy as jnp
from jax.experimental import pallas as pl

N = 50000
DIN = 404
DH = 128
H = 4
DHEAD = DH // H
L = 3
NE = 5

BLK = 512
NPAD = 50176
DINPAD = 512


def _in_proj_kernel(x_ref, w_ref, b_ref, o_ref):
    o_ref[:, :] = jax.nn.relu(
        jnp.dot(x_ref[:, :], w_ref[:, :], preferred_element_type=jnp.float32)
        + b_ref[0, :]
    )


def _qkv_kernel(h_ref, wq_ref, bq_ref, wk_ref, bk_ref, wv_ref, bv_ref,
                abd_ref, mbd_ref, q_ref, ke_ref, ve_ref):
    h = h_ref[:, :]
    q_ref[:, :] = (
        jnp.dot(h, wq_ref[:, :], preferred_element_type=jnp.float32) + bq_ref[0, :]
    )
    k = jnp.dot(h, wk_ref[:, :], preferred_element_type=jnp.float32) + bk_ref[0, :]
    v = jnp.dot(h, wv_ref[:, :], preferred_element_type=jnp.float32) + bv_ref[0, :]
    for e in range(NE):
        ke_ref[e, :, :] = jnp.dot(
            k, abd_ref[e, :, :], preferred_element_type=jnp.float32)
        ve_ref[e, :, :] = jnp.dot(
            v, mbd_ref[e, :, :], preferred_element_type=jnp.float32)


def _post_kernel(h_ref, agg_ref, wa_ref, ba_ref, g_ref, b_ref, beta_ref, o_ref):
    h = h_ref[:, :]
    out = (
        jnp.dot(jax.nn.gelu(agg_ref[:, :]), wa_ref[:, :],
                preferred_element_type=jnp.float32)
        + ba_ref[0, :]
    )
    beta = beta_ref[0, 0]
    out = beta * out + (1.0 - beta) * h
    y = h + out
    mu = jnp.mean(y, axis=-1, keepdims=True)
    var = jnp.mean((y - mu) * (y - mu), axis=-1, keepdims=True)
    o_ref[:, :] = (y - mu) / jnp.sqrt(var + 1e-5) * g_ref[0, :] + b_ref[0, :]


def _nodehead_kernel(h_ref, w1_ref, b1_ref, w2_ref, o_ref):
    t = jax.nn.relu(
        jnp.dot(h_ref[:, :], w1_ref[:, :], preferred_element_type=jnp.float32)
        + b1_ref[0, :]
    )
    o_ref[:, :] = jnp.dot(t, w2_ref[:, :], preferred_element_type=jnp.float32)


def _row_spec():
    return pl.BlockSpec((BLK, DH), lambda i: (i, 0))


def _full2(shape):
    return pl.BlockSpec(shape, lambda i: (0, 0))


def _full3(shape):
    return pl.BlockSpec(shape, lambda i: (0, 0, 0))


_GRID = (NPAD // BLK,)


def _in_proj(xp, w, b):
    return pl.pallas_call(
        _in_proj_kernel,
        grid=_GRID,
        in_specs=[
            pl.BlockSpec((BLK, DINPAD), lambda i: (i, 0)),
            _full2((DINPAD, DH)),
            _full2((1, DH)),
        ],
        out_specs=_row_spec(),
        out_shape=jax.ShapeDtypeStruct((NPAD, DH), jnp.float32),
    )(xp, w, b)


def _qkv(h, wq, bq, wk, bk, wv, bv, abd, mbd):
    return pl.pallas_call(
        _qkv_kernel,
        grid=_GRID,
        in_specs=[
            _row_spec(),
            _full2((DH, DH)), _full2((1, DH)),
            _full2((DH, DH)), _full2((1, DH)),
            _full2((DH, DH)), _full2((1, DH)),
            _full3((NE, DH, DH)),
            _full3((NE, DH, DH)),
        ],
        out_specs=[
            _row_spec(),
            pl.BlockSpec((NE, BLK, DH), lambda i: (0, i, 0)),
            pl.BlockSpec((NE, BLK, DH), lambda i: (0, i, 0)),
        ],
        out_shape=[
            jax.ShapeDtypeStruct((NPAD, DH), jnp.float32),
            jax.ShapeDtypeStruct((NE, NPAD, DH), jnp.float32),
            jax.ShapeDtypeStruct((NE, NPAD, DH), jnp.float32),
        ],
    )(h, wq, bq, wk, bk, wv, bv, abd, mbd)


def _post(h, agg, wa, ba, g, b, beta):
    return pl.pallas_call(
        _post_kernel,
        grid=_GRID,
        in_specs=[
            _row_spec(), _row_spec(),
            _full2((DH, DH)), _full2((1, DH)),
            _full2((1, DH)), _full2((1, DH)),
            _full2((1, 1)),
        ],
        out_specs=_row_spec(),
        out_shape=jax.ShapeDtypeStruct((NPAD, DH), jnp.float32),
    )(h, agg, wa, ba, g, b, beta)


def _nodehead(h, w1, b1, w2):
    return pl.pallas_call(
        _nodehead_kernel,
        grid=_GRID,
        in_specs=[
            _row_spec(),
            _full2((DH, DH)), _full2((1, DH)),
            _full2((DH, DH)),
        ],
        out_specs=_row_spec(),
        out_shape=jax.ShapeDtypeStruct((NPAD, DH), jnp.float32),
    )(h, w1, b1, w2)


def _block_diag(mats):
    out = jnp.zeros((DH, DH), mats.dtype)
    for hh in range(H):
        out = out.at[hh * DHEAD:(hh + 1) * DHEAD,
                     hh * DHEAD:(hh + 1) * DHEAD].set(mats[hh])
    return out


@jax.jit
def kernel(x, edge_index_contains, edge_index_items, edge_index_refers_to,
           edge_index_logic, edge_index_additional, W_in, b_in, Wk, bk, Wq, bq,
           Wv, bv, Wa, ba, a_rel, m_rel, p_rel, skip, ln_g, ln_b, gW1, gb1,
           gW2, gb2, lW1, lb1, lW2, lb2):
    edges = [edge_index_contains, edge_index_items, edge_index_refers_to,
             edge_index_logic, edge_index_additional]

    a_bd = jax.vmap(jax.vmap(_block_diag))(a_rel)
    m_bd = jax.vmap(jax.vmap(_block_diag))(m_rel)

    xp = jnp.pad(x, ((0, NPAD - N), (0, DINPAD - DIN)))
    w_in_p = jnp.pad(W_in, ((0, DINPAD - DIN), (0, 0)))
    h = _in_proj(xp, w_in_p, b_in.reshape(1, DH))

    scale = p_rel / np.sqrt(DHEAD)

    E = edges[0].shape[1]
    src_all = jnp.concatenate([edges[e][0] + e * NPAD for e in range(NE)])
    dst_all = jnp.concatenate([edges[e][1] for e in range(NE)])
    seg2 = jnp.concatenate([edges[e][1] + e * N for e in range(NE)])

    for l in range(L):
        q, ke, ve = _qkv(
            h,
            Wq[l], bq[l].reshape(1, DH),
            Wk[l], bk[l].reshape(1, DH),
            Wv[l], bv[l].reshape(1, DH),
            a_bd[l], m_bd[l],
        )
        ke2 = ke.reshape(NE * NPAD, DH)
        ve2 = ve.reshape(NE * NPAD, DH)
        scale_all = jnp.repeat(scale[l][:, None, :], E, axis=1).reshape(NE * E, H)
        qd = q[dst_all]
        ks = ke2[src_all]
        alpha = (qd * ks).reshape(-1, H, DHEAD).sum(-1) * scale_all
        amax = jax.ops.segment_max(alpha, seg2, num_segments=NE * N)
        amax = jnp.where(jnp.isfinite(amax), amax, 0.0)
        ex = jnp.exp(alpha - amax[seg2])
        denom = jax.ops.segment_sum(ex, seg2, num_segments=NE * N)
        p = ex / (denom[seg2] + 1e-16)
        msg = ve2[src_all].reshape(-1, H, DHEAD) * p[:, :, None]
        agg = jax.ops.segment_sum(msg, dst_all, num_segments=N)
        agg2 = jnp.pad(agg.reshape(N, DH), ((0, NPAD - N), (0, 0)))
        beta = jax.nn.sigmoid(skip[l]).reshape(1, 1)
        h = _post(h, agg2, Wa[l], ba[l].reshape(1, DH),
                  ln_g[l].reshape(1, DH), ln_b[l].reshape(1, DH), beta)

    hN = h[:N]
    mean_pool = jnp.mean(hN, axis=0, keepdims=True)
    max_pool = jnp.max(hN, axis=0, keepdims=True)
    g = jnp.concatenate([mean_pool, max_pool], axis=-1)
    vl = (jax.nn.relu(g @ gW1 + gb1) @ gW2 + gb2).reshape(-1)

    w1p = jnp.pad(lW1, ((0, 0), (0, DH - DH // 2)))
    b1p = jnp.pad(lb1, (0, DH - DH // 2)).reshape(1, DH)
    w2p = jnp.pad(lW2, ((0, DH - DH // 2), (0, DH - 1)))
    nl_full = _nodehead(h, w1p, b1p, w2p)
    nl = nl_full[:N, 0] + lb2[0]

    return (jax.nn.sigmoid(vl), vl, jax.nn.sigmoid(nl), nl)


# speedup vs baseline: 1.2032x; 1.2032x over previous
import numpy as np
import jax
import jax.numpy as jnp
from jax.experimental import pallas as pl

N = 50000
DIN = 404
DH = 128
H = 4
DHEAD = DH // H
L = 3
NE = 5

BLK = 512
NPAD = 50176
DINPAD = 512


def _in_proj_kernel(x_ref, w_ref, b_ref, o_ref):
    o_ref[:, :] = jax.nn.relu(
        jnp.dot(x_ref[:, :], w_ref[:, :], preferred_element_type=jnp.float32)
        + b_ref[0, :]
    )


def _qkv_kernel(h_ref, wq_ref, bq_ref, wk_ref, bk_ref, wv_ref, bv_ref,
                abd_ref, mbd_ref, q_ref, ke_ref, ve_ref):
    h = h_ref[:, :]
    q_ref[:, :] = (
        jnp.dot(h, wq_ref[:, :], preferred_element_type=jnp.float32) + bq_ref[0, :]
    )
    k = jnp.dot(h, wk_ref[:, :], preferred_element_type=jnp.float32) + bk_ref[0, :]
    v = jnp.dot(h, wv_ref[:, :], preferred_element_type=jnp.float32) + bv_ref[0, :]
    for e in range(NE):
        ke_ref[e, :, :] = jnp.dot(
            k, abd_ref[e, :, :], preferred_element_type=jnp.float32)
        ve_ref[e, :, :] = jnp.dot(
            v, mbd_ref[e, :, :], preferred_element_type=jnp.float32)


def _post_kernel(h_ref, agg_ref, wa_ref, ba_ref, g_ref, b_ref, beta_ref, o_ref):
    h = h_ref[:, :]
    out = (
        jnp.dot(jax.nn.gelu(agg_ref[:, :]), wa_ref[:, :],
                preferred_element_type=jnp.float32)
        + ba_ref[0, :]
    )
    beta = beta_ref[0, 0]
    out = beta * out + (1.0 - beta) * h
    y = h + out
    mu = jnp.mean(y, axis=-1, keepdims=True)
    var = jnp.mean((y - mu) * (y - mu), axis=-1, keepdims=True)
    o_ref[:, :] = (y - mu) / jnp.sqrt(var + 1e-5) * g_ref[0, :] + b_ref[0, :]


def _nodehead_kernel(h_ref, w1_ref, b1_ref, w2_ref, o_ref):
    t = jax.nn.relu(
        jnp.dot(h_ref[:, :], w1_ref[:, :], preferred_element_type=jnp.float32)
        + b1_ref[0, :]
    )
    o_ref[:, :] = jnp.dot(t, w2_ref[:, :], preferred_element_type=jnp.float32)


def _row_spec():
    return pl.BlockSpec((BLK, DH), lambda i: (i, 0))


def _full2(shape):
    return pl.BlockSpec(shape, lambda i: (0, 0))


def _full3(shape):
    return pl.BlockSpec(shape, lambda i: (0, 0, 0))


_GRID = (NPAD // BLK,)


def _in_proj(xp, w, b):
    return pl.pallas_call(
        _in_proj_kernel,
        grid=_GRID,
        in_specs=[
            pl.BlockSpec((BLK, DINPAD), lambda i: (i, 0)),
            _full2((DINPAD, DH)),
            _full2((1, DH)),
        ],
        out_specs=_row_spec(),
        out_shape=jax.ShapeDtypeStruct((NPAD, DH), jnp.float32),
    )(xp, w, b)


def _qkv(h, wq, bq, wk, bk, wv, bv, abd, mbd):
    return pl.pallas_call(
        _qkv_kernel,
        grid=_GRID,
        in_specs=[
            _row_spec(),
            _full2((DH, DH)), _full2((1, DH)),
            _full2((DH, DH)), _full2((1, DH)),
            _full2((DH, DH)), _full2((1, DH)),
            _full3((NE, DH, DH)),
            _full3((NE, DH, DH)),
        ],
        out_specs=[
            _row_spec(),
            pl.BlockSpec((NE, BLK, DH), lambda i: (0, i, 0)),
            pl.BlockSpec((NE, BLK, DH), lambda i: (0, i, 0)),
        ],
        out_shape=[
            jax.ShapeDtypeStruct((NPAD, DH), jnp.float32),
            jax.ShapeDtypeStruct((NE, NPAD, DH), jnp.float32),
            jax.ShapeDtypeStruct((NE, NPAD, DH), jnp.float32),
        ],
    )(h, wq, bq, wk, bk, wv, bv, abd, mbd)


def _post(h, agg, wa, ba, g, b, beta):
    return pl.pallas_call(
        _post_kernel,
        grid=_GRID,
        in_specs=[
            _row_spec(), _row_spec(),
            _full2((DH, DH)), _full2((1, DH)),
            _full2((1, DH)), _full2((1, DH)),
            _full2((1, 1)),
        ],
        out_specs=_row_spec(),
        out_shape=jax.ShapeDtypeStruct((NPAD, DH), jnp.float32),
    )(h, agg, wa, ba, g, b, beta)


def _nodehead(h, w1, b1, w2):
    return pl.pallas_call(
        _nodehead_kernel,
        grid=_GRID,
        in_specs=[
            _row_spec(),
            _full2((DH, DH)), _full2((1, DH)),
            _full2((DH, DH)),
        ],
        out_specs=_row_spec(),
        out_shape=jax.ShapeDtypeStruct((NPAD, DH), jnp.float32),
    )(h, w1, b1, w2)


def _block_diag(mats):
    out = jnp.zeros((DH, DH), mats.dtype)
    for hh in range(H):
        out = out.at[hh * DHEAD:(hh + 1) * DHEAD,
                     hh * DHEAD:(hh + 1) * DHEAD].set(mats[hh])
    return out


@jax.jit
def kernel(x, edge_index_contains, edge_index_items, edge_index_refers_to,
           edge_index_logic, edge_index_additional, W_in, b_in, Wk, bk, Wq, bq,
           Wv, bv, Wa, ba, a_rel, m_rel, p_rel, skip, ln_g, ln_b, gW1, gb1,
           gW2, gb2, lW1, lb1, lW2, lb2):
    edges = [edge_index_contains, edge_index_items, edge_index_refers_to,
             edge_index_logic, edge_index_additional]

    a_bd = jax.vmap(jax.vmap(_block_diag))(a_rel)
    m_bd = jax.vmap(jax.vmap(_block_diag))(m_rel)

    xp = jnp.pad(x, ((0, NPAD - N), (0, DINPAD - DIN)))
    w_in_p = jnp.pad(W_in, ((0, DINPAD - DIN), (0, 0)))
    h = _in_proj(xp, w_in_p, b_in.reshape(1, DH))

    scale = p_rel / np.sqrt(DHEAD)

    for l in range(L):
        q, ke, ve = _qkv(
            h,
            Wq[l], bq[l].reshape(1, DH),
            Wk[l], bk[l].reshape(1, DH),
            Wv[l], bv[l].reshape(1, DH),
            a_bd[l], m_bd[l],
        )
        agg = jnp.zeros((N, H, DHEAD), jnp.float32)
        for e in range(NE):
            src = edges[e][0]
            dst = edges[e][1]
            qd = q[dst]
            ks = ke[e][src]
            alpha = (qd * ks).reshape(-1, H, DHEAD).sum(-1) * scale[l, e]
            amax = jax.ops.segment_max(alpha, dst, num_segments=N)
            amax = jnp.where(jnp.isfinite(amax), amax, 0.0)
            ex = jnp.exp(alpha - amax[dst])
            denom = jax.ops.segment_sum(ex, dst, num_segments=N)
            p = ex / (denom[dst] + 1e-16)
            msg = ve[e][src].reshape(-1, H, DHEAD) * p[:, :, None]
            agg = agg + jax.ops.segment_sum(msg, dst, num_segments=N)
        agg2 = jnp.pad(agg.reshape(N, DH), ((0, NPAD - N), (0, 0)))
        beta = jax.nn.sigmoid(skip[l]).reshape(1, 1)
        h = _post(h, agg2, Wa[l], ba[l].reshape(1, DH),
                  ln_g[l].reshape(1, DH), ln_b[l].reshape(1, DH), beta)

    hN = h[:N]
    mean_pool = jnp.mean(hN, axis=0, keepdims=True)
    max_pool = jnp.max(hN, axis=0, keepdims=True)
    g = jnp.concatenate([mean_pool, max_pool], axis=-1)
    vl = (jax.nn.relu(g @ gW1 + gb1) @ gW2 + gb2).reshape(-1)

    w1p = jnp.pad(lW1, ((0, 0), (0, DH - DH // 2)))
    b1p = jnp.pad(lb1, (0, DH - DH // 2)).reshape(1, DH)
    w2p = jnp.pad(lW2, ((0, DH - DH // 2), (0, DH - 1)))
    nl_full = _nodehead(h, w1p, b1p, w2p)
    nl = nl_full[:N, 0] + lb2[0]

    return (jax.nn.sigmoid(vl), vl, jax.nn.sigmoid(nl), nl)
